# Initial kernel scaffold; baseline (speedup 1.0000x reference)
import jax
import jax.numpy as jnp
from jax.experimental import pallas as pl


def kernel(x, edge_index, batch, W1, b1, W2, b2):
    raise NotImplementedError("write your pallas kernel here")


# speedup vs baseline: 13.6244x; 13.6244x over previous
import functools

import jax
import jax.numpy as jnp
from jax import lax
from jax.experimental import pallas as pl
from jax.experimental.pallas import tpu as pltpu
from jax.experimental.pallas import tpu_sc as plsc

N = 10000
E = 320000
D = 128
G = 128
NP = 10240
NC = 2
NS = 16
NW = NC * NS
EPT = E // NW
ROWS_PT = NP // NW
CH = 80
NCH = EPT // CH
DR = NP // D

_sc_mesh = plsc.VectorSubcoreMesh(core_axis_name="c", subcore_axis_name="s")
_sc_params = pltpu.CompilerParams(needs_layout_passes=False)


@functools.partial(
    pl.kernel,
    out_type=jax.ShapeDtypeStruct((NW, NP), jnp.float32),
    mesh=_sc_mesh,
    compiler_params=_sc_params,
    scratch_types=[
        pltpu.VMEM((NP,), jnp.float32),
        pltpu.VMEM((EPT,), jnp.int32),
    ],
)
def _deg_kernel(ecol_hbm, zeros_hbm, degp_out, deg_v, ecol_v):
    c = lax.axis_index("c")
    s = lax.axis_index("s")
    wid = c * NS + s
    pltpu.sync_copy(zeros_hbm.at[pl.ds(0, NP)], deg_v)
    pltpu.sync_copy(ecol_hbm.at[pl.ds(wid * EPT, EPT)], ecol_v)
    ones = jnp.full((16,), 1.0, dtype=jnp.float32)

    def body(i, carry):
        idx = ecol_v[pl.ds(i * 16, 16)]
        plsc.addupdate_scatter(deg_v, [idx], ones)
        return carry

    lax.fori_loop(0, EPT // 16, body, 0)
    pltpu.sync_copy(deg_v, degp_out.at[wid])


def _xw_body(x_ref, w1_ref, degp_ref, y_ref, dinv_ref):
    deg = jnp.sum(degp_ref[...], axis=0) + 1.0
    dinv = lax.rsqrt(deg)
    xw = jnp.dot(x_ref[...], w1_ref[...], preferred_element_type=jnp.float32)
    y_ref[...] = xw * dinv
    dinv_ref[...] = dinv


_BB = 1280
_xw_call = pl.pallas_call(
    _xw_body,
    grid=(NP // _BB,),
    in_specs=[
        pl.BlockSpec((_BB, D), lambda i: (i, 0)),
        pl.BlockSpec((D, D), lambda i: (0, 0)),
        pl.BlockSpec((NW, _BB, 1), lambda i: (0, i, 0)),
    ],
    out_specs=[
        pl.BlockSpec((_BB, D), lambda i: (i, 0)),
        pl.BlockSpec((_BB, 1), lambda i: (i, 0)),
    ],
    out_shape=[
        jax.ShapeDtypeStruct((NP, D), jnp.float32),
        jax.ShapeDtypeStruct((NP, 1), jnp.float32),
    ],
)


@functools.partial(
    pl.kernel,
    out_type=jax.ShapeDtypeStruct((NC, NP, D), jnp.float32),
    mesh=_sc_mesh,
    compiler_params=_sc_params,
    scratch_types=[
        pltpu.VMEM_SHARED((NP, D), jnp.float32),
        pltpu.VMEM((CH,), jnp.int32),
        pltpu.VMEM((CH,), jnp.int32),
        pltpu.VMEM((CH, D), jnp.float32),
        pltpu.VMEM((64, D), jnp.float32),
        pltpu.SemaphoreType.DMA,
    ],
)
def _scatter_kernel(y_hbm, erow_hbm, ecol_hbm, zeros_hbm, s_out,
                    shared, ir, ic, rb, zb, sem):
    c = lax.axis_index("c")
    s = lax.axis_index("s")
    pltpu.sync_copy(zeros_hbm.at[pl.ds(0, 64)], zb)
    for q in range(10):
        pltpu.sync_copy(zb, shared.at[pl.ds(s * 640 + q * 64, 64)])
    plsc.subcore_barrier()

    base0 = (c * NS + s) * EPT

    def body(i, carry):
        base = base0 + i * CH
        pltpu.sync_copy(erow_hbm.at[pl.ds(base, CH)], ir)
        pltpu.sync_copy(ecol_hbm.at[pl.ds(base, CH)], ic)
        pltpu.async_copy(y_hbm.at[ir], rb, sem).wait()
        pltpu.sync_copy(rb, shared.at[ic], add=True)
        return carry

    lax.fori_loop(0, NCH, body, 0)
    plsc.subcore_barrier()

    pltpu.sync_copy(shared.at[pl.ds(s * 640, 640)],
                    s_out.at[c, pl.ds(s * 640, 640)])


_RCH = 64


@functools.partial(
    pl.kernel,
    out_type=jax.ShapeDtypeStruct((NW, G * D), jnp.float32),
    mesh=_sc_mesh,
    compiler_params=_sc_params,
    scratch_types=[
        pltpu.VMEM((G * D,), jnp.float32),
        pltpu.VMEM((_RCH * D,), jnp.float32),
        pltpu.VMEM((_RCH * D,), jnp.float32),
        pltpu.VMEM((_RCH * D,), jnp.float32),
        pltpu.VMEM((_RCH,), jnp.float32),
        pltpu.VMEM((_RCH,), jnp.int32),
        pltpu.VMEM((D,), jnp.float32),
    ],
)
def _segmax_kernel(s_hbm, y_hbm, dinv_hbm, batch_hbm, b1_hbm, zeros_hbm, hp_out,
                   hp, s0b, s1b, yb, dvb, btb, b1b):
    c = lax.axis_index("c")
    s = lax.axis_index("s")
    wid = c * NS + s
    pltpu.sync_copy(zeros_hbm.at[pl.ds(0, G * D)], hp)
    pltpu.sync_copy(b1_hbm, b1b)
    tbase = wid * ROWS_PT

    def chunk_body(ch, carry):
        base = tbase + ch * _RCH
        pltpu.sync_copy(s_hbm.at[0, pl.ds(base * D, _RCH * D)], s0b)
        pltpu.sync_copy(s_hbm.at[1, pl.ds(base * D, _RCH * D)], s1b)
        pltpu.sync_copy(y_hbm.at[pl.ds(base * D, _RCH * D)], yb)
        pltpu.sync_copy(dinv_hbm.at[pl.ds(base, _RCH)], dvb)
        pltpu.sync_copy(batch_hbm.at[pl.ds(base, _RCH)], btb)
        ngrp = jnp.clip((N - base) // 16, 0, _RCH // 16)

        def grp_body(gi, carry2):
            bt16 = btb[pl.ds(gi * 16, 16)]
            dv16 = dvb[pl.ds(gi * 16, 16)]
            gofs = bt16 * D
            for k in range(16):
                r = gi * 16 + k
                g = gofs[k]
                dv = dv16[k]
                for j in range(8):
                    rsl = pl.ds(r * D + j * 16, 16)
                    hv = (s0b[rsl] + s1b[rsl] + yb[rsl]) * dv + b1b[pl.ds(j * 16, 16)]
                    hv = jnp.maximum(hv, 0.0)
                    hsl = pl.ds(g + j * 16, 16)
                    hp[hsl] = jnp.maximum(hp[hsl], hv)
            return carry2

        lax.fori_loop(0, ngrp, grp_body, 0)
        return carry

    lax.fori_loop(0, ROWS_PT // _RCH, chunk_body, 0)
    pltpu.sync_copy(hp, hp_out.at[wid])


def _final_body(hp_ref, w2_ref, b2_ref, out_ref):
    hp = jnp.max(hp_ref[...], axis=0)
    o = jnp.dot(hp, w2_ref[...], preferred_element_type=jnp.float32) + b2_ref[...]
    m = jnp.max(o, axis=-1, keepdims=True)
    z = o - m
    out_ref[...] = z - jnp.log(jnp.sum(jnp.exp(z), axis=-1, keepdims=True))


_final_call = pl.pallas_call(
    _final_body,
    out_shape=jax.ShapeDtypeStruct((G, 2), jnp.float32),
)


def kernel(x, edge_index, batch, W1, b1, W2, b2):
    erow = edge_index[0]
    ecol = edge_index[1]
    x_pad = jnp.pad(x, ((0, NP - N), (0, 0)))
    batch_pad = jnp.pad(batch, (0, NP - N))
    zeros = jnp.zeros((ROWS_PT, D), jnp.float32)
    zeros_flat = jnp.zeros((G * D,), jnp.float32)
    degp = _deg_kernel(ecol, zeros_flat)
    y, dinv = _xw_call(x_pad, W1, degp.reshape(NW, NP, 1))
    s_part = _scatter_kernel(y, erow, ecol, zeros)
    hp_parts = _segmax_kernel(s_part.reshape(NC, NP * D), y.reshape(NP * D),
                              dinv.reshape(NP), batch_pad, b1, zeros_flat)
    return _final_call(hp_parts.reshape(NW, G, D), W2, b2.reshape(1, 2))


# speedup vs baseline: 18.5510x; 1.3616x over previous
import functools

import jax
import jax.numpy as jnp
from jax import lax
from jax.experimental import pallas as pl
from jax.experimental.pallas import tpu as pltpu
from jax.experimental.pallas import tpu_sc as plsc

N = 10000
E = 320000
D = 128
G = 128
NP = 10240
NC = 2
NS = 16
NW = NC * NS
EPT = E // NW
ROWS_PT = NP // NW
CH = 80
NCH = EPT // CH
DR = NP // D

_sc_mesh = plsc.VectorSubcoreMesh(core_axis_name="c", subcore_axis_name="s")
_sc_params = pltpu.CompilerParams(needs_layout_passes=False)


@functools.partial(
    pl.kernel,
    out_type=jax.ShapeDtypeStruct((NW, NP), jnp.float32),
    mesh=_sc_mesh,
    compiler_params=_sc_params,
    scratch_types=[
        pltpu.VMEM((NP,), jnp.float32),
        pltpu.VMEM((EPT,), jnp.int32),
    ],
)
def _deg_kernel(ecol_hbm, zeros_hbm, degp_out, deg_v, ecol_v):
    c = lax.axis_index("c")
    s = lax.axis_index("s")
    wid = c * NS + s
    pltpu.sync_copy(zeros_hbm.at[pl.ds(0, NP)], deg_v)
    pltpu.sync_copy(ecol_hbm.at[pl.ds(wid * EPT, EPT)], ecol_v)
    ones = jnp.full((16,), 1.0, dtype=jnp.float32)

    def body(i, carry):
        idx = ecol_v[pl.ds(i * 16, 16)]
        plsc.addupdate_scatter(deg_v, [idx], ones)
        return carry

    lax.fori_loop(0, EPT // 16, body, 0)
    pltpu.sync_copy(deg_v, degp_out.at[wid])


def _xw_body(x_ref, w1_ref, degp_ref, y_ref, dinv_ref):
    deg = jnp.sum(degp_ref[...], axis=0) + 1.0
    dinv = lax.rsqrt(deg)
    xw = jnp.dot(x_ref[...], w1_ref[...], preferred_element_type=jnp.float32)
    y_ref[...] = xw * dinv
    dinv_ref[...] = dinv


_BB = 1280
_xw_call = pl.pallas_call(
    _xw_body,
    grid=(NP // _BB,),
    in_specs=[
        pl.BlockSpec((_BB, D), lambda i: (i, 0)),
        pl.BlockSpec((D, D), lambda i: (0, 0)),
        pl.BlockSpec((NW, _BB, 1), lambda i: (0, i, 0)),
    ],
    out_specs=[
        pl.BlockSpec((_BB, D), lambda i: (i, 0)),
        pl.BlockSpec((_BB, 1), lambda i: (i, 0)),
    ],
    out_shape=[
        jax.ShapeDtypeStruct((NP, D), jnp.float32),
        jax.ShapeDtypeStruct((NP, 1), jnp.float32),
    ],
)


@functools.partial(
    pl.kernel,
    out_type=jax.ShapeDtypeStruct((NC, NP, D), jnp.float32),
    mesh=_sc_mesh,
    compiler_params=_sc_params,
    scratch_types=[
        pltpu.VMEM_SHARED((NP, D), jnp.float32),
        pltpu.VMEM((EPT,), jnp.int32),
        pltpu.VMEM((NCH, CH), jnp.int32),
        pltpu.VMEM((CH, D), jnp.float32),
        pltpu.VMEM((CH, D), jnp.float32),
        pltpu.SemaphoreType.DMA,
        pltpu.SemaphoreType.DMA,
        pltpu.SemaphoreType.DMA,
        pltpu.SemaphoreType.DMA,
    ],
)
def _scatter_kernel(y_hbm, erow_hbm, ecol_hbm, zeros_hbm, s_out,
                    shared, ir_all, ic_all, rb0, rb1, gsem0, gsem1, ssem0, ssem1):
    c = lax.axis_index("c")
    s = lax.axis_index("s")
    wid = c * NS + s
    pltpu.sync_copy(zeros_hbm, rb0)
    for q in range(8):
        pltpu.sync_copy(rb0, shared.at[pl.ds(s * 640 + q * CH, CH)])
    pltpu.sync_copy(erow_hbm.at[pl.ds(wid * EPT, EPT)], ir_all)
    pltpu.sync_copy(ecol_hbm.at[wid], ic_all)
    plsc.subcore_barrier()

    def gather(i, rb, sem):
        return pltpu.async_copy(y_hbm.at[ir_all.at[pl.ds(i * CH, CH)]], rb, sem)

    def scatter(i, rb, sem):
        return pltpu.async_copy(rb, shared.at[ic_all.at[i]], sem, add=True)

    gather(0, rb0, gsem0)
    gather(1, rb1, gsem1)
    pltpu.make_async_copy(y_hbm.at[ir_all.at[pl.ds(0, CH)]], rb0, gsem0).wait()
    scatter(0, rb0, ssem0)
    pltpu.make_async_copy(y_hbm.at[ir_all.at[pl.ds(0, CH)]], rb1, gsem1).wait()
    pltpu.make_async_copy(rb0, shared.at[ic_all.at[0]], ssem0).wait()
    gather(2, rb0, gsem0)
    scatter(1, rb1, ssem1)

    def body(k, carry):
        i0 = 2 * k
        pltpu.make_async_copy(y_hbm.at[ir_all.at[pl.ds(0, CH)]], rb0, gsem0).wait()
        pltpu.make_async_copy(rb1, shared.at[ic_all.at[0]], ssem1).wait()
        gather(i0 + 1, rb1, gsem1)
        scatter(i0, rb0, ssem0)
        pltpu.make_async_copy(y_hbm.at[ir_all.at[pl.ds(0, CH)]], rb1, gsem1).wait()
        pltpu.make_async_copy(rb0, shared.at[ic_all.at[0]], ssem0).wait()
        gather(i0 + 2, rb0, gsem0)
        scatter(i0 + 1, rb1, ssem1)
        return carry

    lax.fori_loop(1, (NCH - 1) // 2, body, 0)
    pltpu.make_async_copy(y_hbm.at[ir_all.at[pl.ds(0, CH)]], rb0, gsem0).wait()
    pltpu.make_async_copy(rb1, shared.at[ic_all.at[0]], ssem1).wait()
    scatter(NCH - 1, rb0, ssem0)
    pltpu.make_async_copy(rb0, shared.at[ic_all.at[0]], ssem0).wait()
    plsc.subcore_barrier()

    pltpu.sync_copy(shared.at[pl.ds(s * 640, 640)],
                    s_out.at[c, pl.ds(s * 640, 640)])


_RCH = 64


@functools.partial(
    pl.kernel,
    out_type=jax.ShapeDtypeStruct((NW, G * D), jnp.float32),
    mesh=_sc_mesh,
    compiler_params=_sc_params,
    scratch_types=[
        pltpu.VMEM((G * D,), jnp.float32),
        pltpu.VMEM((_RCH * D,), jnp.float32),
        pltpu.VMEM((_RCH * D,), jnp.float32),
        pltpu.VMEM((_RCH * D,), jnp.float32),
        pltpu.VMEM((_RCH,), jnp.float32),
        pltpu.VMEM((_RCH,), jnp.int32),
        pltpu.VMEM((D,), jnp.float32),
    ],
)
def _segmax_kernel(s_hbm, y_hbm, dinv_hbm, batch_hbm, b1_hbm, zeros_hbm, hp_out,
                   hp, s0b, s1b, yb, dvb, btb, b1b):
    c = lax.axis_index("c")
    s = lax.axis_index("s")
    wid = c * NS + s
    pltpu.sync_copy(zeros_hbm.at[pl.ds(0, G * D)], hp)
    pltpu.sync_copy(b1_hbm, b1b)
    tbase = wid * ROWS_PT

    def chunk_body(ch, carry):
        base = tbase + ch * _RCH
        pltpu.sync_copy(s_hbm.at[0, pl.ds(base * D, _RCH * D)], s0b)
        pltpu.sync_copy(s_hbm.at[1, pl.ds(base * D, _RCH * D)], s1b)
        pltpu.sync_copy(y_hbm.at[pl.ds(base * D, _RCH * D)], yb)
        pltpu.sync_copy(dinv_hbm.at[pl.ds(base, _RCH)], dvb)
        pltpu.sync_copy(batch_hbm.at[pl.ds(base, _RCH)], btb)
        ngrp = jnp.clip((N - base) // 16, 0, _RCH // 16)

        def grp_body(gi, carry2):
            bt16 = btb[pl.ds(gi * 16, 16)]
            dv16 = dvb[pl.ds(gi * 16, 16)]
            gofs = bt16 * D
            for k in range(16):
                r = gi * 16 + k
                g = gofs[k]
                dv = dv16[k]
                for j in range(8):
                    rsl = pl.ds(r * D + j * 16, 16)
                    hv = (s0b[rsl] + s1b[rsl] + yb[rsl]) * dv + b1b[pl.ds(j * 16, 16)]
                    hv = jnp.maximum(hv, 0.0)
                    hsl = pl.ds(g + j * 16, 16)
                    hp[hsl] = jnp.maximum(hp[hsl], hv)
            return carry2

        lax.fori_loop(0, ngrp, grp_body, 0)
        return carry

    lax.fori_loop(0, ROWS_PT // _RCH, chunk_body, 0)
    pltpu.sync_copy(hp, hp_out.at[wid])


def _final_body(hp_ref, w2_ref, b2_ref, out_ref):
    hp = jnp.max(hp_ref[...], axis=0)
    o = jnp.dot(hp, w2_ref[...], preferred_element_type=jnp.float32) + b2_ref[...]
    m = jnp.max(o, axis=-1, keepdims=True)
    z = o - m
    out_ref[...] = z - jnp.log(jnp.sum(jnp.exp(z), axis=-1, keepdims=True))


_final_call = pl.pallas_call(
    _final_body,
    out_shape=jax.ShapeDtypeStruct((G, 2), jnp.float32),
)


def kernel(x, edge_index, batch, W1, b1, W2, b2):
    erow = edge_index[0]
    ecol = edge_index[1]
    x_pad = jnp.pad(x, ((0, NP - N), (0, 0)))
    batch_pad = jnp.pad(batch, (0, NP - N))
    zeros80 = jnp.zeros((CH, D), jnp.float32)
    zeros_flat = jnp.zeros((G * D,), jnp.float32)
    degp = _deg_kernel(ecol, zeros_flat)
    y, dinv = _xw_call(x_pad, W1, degp.reshape(NW, NP, 1))
    s_part = _scatter_kernel(y, erow, ecol.reshape(NW, NCH, CH), zeros80)
    hp_parts = _segmax_kernel(s_part.reshape(NC, NP * D), y.reshape(NP * D),
                              dinv.reshape(NP), batch_pad, b1, zeros_flat)
    return _final_call(hp_parts.reshape(NW, G, D), W2, b2.reshape(1, 2))


# speedup vs baseline: 26.7749x; 1.4433x over previous
import functools

import jax
import jax.numpy as jnp
from jax import lax
from jax.experimental import pallas as pl
from jax.experimental.pallas import tpu as pltpu
from jax.experimental.pallas import tpu_sc as plsc

N = 10000
E = 320000
D = 128
G = 128
NP = 10240
NC = 2
NS = 16
NW = NC * NS
EPT = E // NW
ROWS_PT = NP // NW
CH = 80
NCH = EPT // CH
DR = NP // D

_sc_mesh = plsc.VectorSubcoreMesh(core_axis_name="c", subcore_axis_name="s")
_sc_params = pltpu.CompilerParams(needs_layout_passes=False)


_W = NP // NS


@functools.partial(
    pl.kernel,
    out_type=jax.ShapeDtypeStruct((NC, NP), jnp.float32),
    mesh=_sc_mesh,
    compiler_params=_sc_params,
    scratch_types=[
        pltpu.VMEM_SHARED((NS, NP), jnp.float32),
        pltpu.VMEM((NP,), jnp.float32),
        pltpu.VMEM((EPT,), jnp.int32),
        pltpu.VMEM((NP,), jnp.float32),
    ],
)
def _deg_kernel(ecol_hbm, zeros_hbm, degp_out, shared, deg_v, ecol_v, strip_v):
    c = lax.axis_index("c")
    s = lax.axis_index("s")
    wid = c * NS + s
    pltpu.sync_copy(zeros_hbm.at[pl.ds(0, NP)], deg_v)
    pltpu.sync_copy(ecol_hbm.at[pl.ds(wid * EPT, EPT)], ecol_v)
    ones = jnp.full((16,), 1.0, dtype=jnp.float32)

    def body(i, carry):
        idx = ecol_v[pl.ds(i * 16, 16)]
        plsc.addupdate_scatter(deg_v, [idx], ones)
        return carry

    lax.fori_loop(0, EPT // 16, body, 0)
    pltpu.sync_copy(deg_v, shared.at[s])
    plsc.subcore_barrier()
    for p in range(NS):
        pltpu.sync_copy(shared.at[p, pl.ds(s * _W, _W)],
                        strip_v.at[pl.ds(p * _W, _W)])

    def sum_body(v, carry):
        acc = strip_v[pl.ds(v * 16, 16)]
        for p in range(1, NS):
            acc = acc + strip_v[pl.ds(p * _W + v * 16, 16)]
        deg_v[pl.ds(s * _W + v * 16, 16)] = acc
        return carry

    lax.fori_loop(0, _W // 16, sum_body, 0)
    pltpu.sync_copy(deg_v.at[pl.ds(s * _W, _W)],
                    degp_out.at[c, pl.ds(s * _W, _W)])


def _xw_body(x_ref, w1_ref, degp_ref, y_ref, dinv_ref):
    deg = jnp.sum(degp_ref[...], axis=0) + 1.0
    dinv = lax.rsqrt(deg)
    xw = jnp.dot(x_ref[...], w1_ref[...], preferred_element_type=jnp.float32)
    y_ref[...] = xw * dinv
    dinv_ref[...] = dinv


_BB = 2000
_xw_call = pl.pallas_call(
    _xw_body,
    grid=(N // _BB,),
    in_specs=[
        pl.BlockSpec((_BB, D), lambda i: (i, 0)),
        pl.BlockSpec((D, D), lambda i: (0, 0)),
        pl.BlockSpec((NC, _BB, 1), lambda i: (0, i, 0)),
    ],
    out_specs=[
        pl.BlockSpec((_BB, D), lambda i: (i, 0)),
        pl.BlockSpec((_BB, 1), lambda i: (i, 0)),
    ],
    out_shape=[
        jax.ShapeDtypeStruct((N, D), jnp.float32),
        jax.ShapeDtypeStruct((N, 1), jnp.float32),
    ],
)


@functools.partial(
    pl.kernel,
    out_type=jax.ShapeDtypeStruct((NC, NP, D), jnp.float32),
    mesh=_sc_mesh,
    compiler_params=_sc_params,
    scratch_types=[
        pltpu.VMEM_SHARED((NP, D), jnp.float32),
        pltpu.VMEM((EPT,), jnp.int32),
        pltpu.VMEM((NCH, CH), jnp.int32),
        pltpu.VMEM((CH, D), jnp.float32),
        pltpu.VMEM((CH, D), jnp.float32),
        pltpu.SemaphoreType.DMA,
        pltpu.SemaphoreType.DMA,
        pltpu.SemaphoreType.DMA,
        pltpu.SemaphoreType.DMA,
    ],
)
def _scatter_kernel(y_hbm, erow_hbm, ecol_hbm, zeros_hbm, s_out,
                    shared, ir_all, ic_all, rb0, rb1, gsem0, gsem1, ssem0, ssem1):
    c = lax.axis_index("c")
    s = lax.axis_index("s")
    wid = c * NS + s
    pltpu.sync_copy(zeros_hbm, rb0)
    for q in range(8):
        pltpu.sync_copy(rb0, shared.at[pl.ds(s * 640 + q * CH, CH)])
    pltpu.sync_copy(erow_hbm.at[pl.ds(wid * EPT, EPT)], ir_all)
    pltpu.sync_copy(ecol_hbm.at[wid], ic_all)
    plsc.subcore_barrier()

    def gather(i, rb, sem):
        return pltpu.async_copy(y_hbm.at[ir_all.at[pl.ds(i * CH, CH)]], rb, sem)

    def scatter(i, rb, sem):
        return pltpu.async_copy(rb, shared.at[ic_all.at[i]], sem, add=True)

    gather(0, rb0, gsem0)
    gather(1, rb1, gsem1)
    pltpu.make_async_copy(y_hbm.at[ir_all.at[pl.ds(0, CH)]], rb0, gsem0).wait()
    scatter(0, rb0, ssem0)
    pltpu.make_async_copy(y_hbm.at[ir_all.at[pl.ds(0, CH)]], rb1, gsem1).wait()
    pltpu.make_async_copy(rb0, shared.at[ic_all.at[0]], ssem0).wait()
    gather(2, rb0, gsem0)
    scatter(1, rb1, ssem1)

    def body(k, carry):
        i0 = 2 * k
        pltpu.make_async_copy(y_hbm.at[ir_all.at[pl.ds(0, CH)]], rb0, gsem0).wait()
        pltpu.make_async_copy(rb1, shared.at[ic_all.at[0]], ssem1).wait()
        gather(i0 + 1, rb1, gsem1)
        scatter(i0, rb0, ssem0)
        pltpu.make_async_copy(y_hbm.at[ir_all.at[pl.ds(0, CH)]], rb1, gsem1).wait()
        pltpu.make_async_copy(rb0, shared.at[ic_all.at[0]], ssem0).wait()
        gather(i0 + 2, rb0, gsem0)
        scatter(i0 + 1, rb1, ssem1)
        return carry

    lax.fori_loop(1, (NCH - 1) // 2, body, 0)
    pltpu.make_async_copy(y_hbm.at[ir_all.at[pl.ds(0, CH)]], rb0, gsem0).wait()
    pltpu.make_async_copy(rb1, shared.at[ic_all.at[0]], ssem1).wait()
    scatter(NCH - 1, rb0, ssem0)
    pltpu.make_async_copy(rb0, shared.at[ic_all.at[0]], ssem0).wait()
    plsc.subcore_barrier()

    pltpu.sync_copy(shared.at[pl.ds(s * 640, 640)],
                    s_out.at[c, pl.ds(s * 640, 640)])


_RCH = 80


@functools.partial(
    pl.kernel,
    out_type=jax.ShapeDtypeStruct((NW, G * D), jnp.float32),
    mesh=_sc_mesh,
    compiler_params=_sc_params,
    scratch_types=[
        pltpu.VMEM((G * D,), jnp.float32),
        pltpu.VMEM((_RCH * D,), jnp.float32),
        pltpu.VMEM((_RCH * D,), jnp.float32),
        pltpu.VMEM((_RCH * D,), jnp.float32),
        pltpu.VMEM((ROWS_PT,), jnp.float32),
        pltpu.VMEM((ROWS_PT,), jnp.int32),
        pltpu.VMEM((D,), jnp.float32),
    ],
)
def _segmax_kernel(s_hbm, y_hbm, dinv_hbm, batch_hbm, b1_hbm, zeros_hbm, hp_out,
                   hp, s0b, s1b, yb, dvb, btb, b1b):
    c = lax.axis_index("c")
    s = lax.axis_index("s")
    wid = c * NS + s
    tbase = wid * ROWS_PT
    pltpu.sync_copy(zeros_hbm.at[pl.ds(0, G * D)], hp)
    pltpu.sync_copy(b1_hbm, b1b)
    pltpu.sync_copy(dinv_hbm.at[pl.ds(tbase, ROWS_PT)], dvb)
    pltpu.sync_copy(batch_hbm.at[pl.ds(tbase, ROWS_PT)], btb)
    nch = jnp.clip((N - tbase) // _RCH, 0, ROWS_PT // _RCH)

    def chunk_body(ch, carry):
        base = tbase + ch * _RCH
        pltpu.sync_copy(s_hbm.at[0, pl.ds(base * D, _RCH * D)], s0b)
        pltpu.sync_copy(s_hbm.at[1, pl.ds(base * D, _RCH * D)], s1b)
        pltpu.sync_copy(y_hbm.at[pl.ds(base * D, _RCH * D)], yb)

        def grp_body(gi, carry2):
            bt16 = btb[pl.ds(ch * _RCH + gi * 16, 16)]
            dv16 = dvb[pl.ds(ch * _RCH + gi * 16, 16)]
            gofs = bt16 * D
            rbase = gi * 16 * D
            for k in range(16):
                g = gofs[k]
                dv = dv16[k]
                for j in range(8):
                    rsl = pl.ds(rbase + k * D + j * 16, 16)
                    hv = (s0b[rsl] + s1b[rsl] + yb[rsl]) * dv + b1b[pl.ds(j * 16, 16)]
                    hv = jnp.maximum(hv, 0.0)
                    hsl = pl.ds(g + j * 16, 16)
                    hp[hsl] = jnp.maximum(hp[hsl], hv)
            return carry2

        lax.fori_loop(0, _RCH // 16, grp_body, 0)
        return carry

    lax.fori_loop(0, nch, chunk_body, 0)
    pltpu.sync_copy(hp, hp_out.at[wid])


def _final_body(hp_ref, w2_ref, b2_ref, out_ref):
    hp = jnp.max(hp_ref[...], axis=0)
    o = jnp.dot(hp, w2_ref[...], preferred_element_type=jnp.float32) + b2_ref[...]
    m = jnp.max(o, axis=-1, keepdims=True)
    z = o - m
    out_ref[...] = z - jnp.log(jnp.sum(jnp.exp(z), axis=-1, keepdims=True))


_final_call = pl.pallas_call(
    _final_body,
    out_shape=jax.ShapeDtypeStruct((G, 2), jnp.float32),
)


def kernel(x, edge_index, batch, W1, b1, W2, b2):
    erow = edge_index[0]
    ecol = edge_index[1]
    batch_pad = jnp.pad(batch, (0, NP - N))
    zeros80 = jnp.zeros((CH, D), jnp.float32)
    zeros_flat = jnp.zeros((G * D,), jnp.float32)
    degp = _deg_kernel(ecol, zeros_flat)
    y, dinv = _xw_call(x, W1, degp.reshape(NC, NP, 1))
    s_part = _scatter_kernel(y, erow, ecol.reshape(NW, NCH, CH), zeros80)
    dinv_pad = jnp.pad(dinv.reshape(N), (0, NP - N))
    hp_parts = _segmax_kernel(s_part.reshape(NC, NP * D), y.reshape(N * D),
                              dinv_pad, batch_pad, b1, zeros_flat)
    return _final_call(hp_parts.reshape(NW, G, D), W2, b2.reshape(1, 2))
